# Initial kernel scaffold; baseline (speedup 1.0000x reference)
import jax
import jax.numpy as jnp
from jax.experimental import pallas as pl


def kernel(table, ln1_g, ln1_b, W1, b1, ln2_g, ln2_b, W2, b2):
    raise NotImplementedError("write your pallas kernel here")


# speedup vs baseline: 47.5743x; 47.5743x over previous
import functools

import jax
import jax.numpy as jnp
from jax.experimental import pallas as pl
from jax.experimental.pallas import tpu as pltpu

_BIG = 1e30
_HI = jax.lax.Precision.HIGHEST


def _dot(a, b, dims):
    return jax.lax.dot_general(a, b, (dims, ((), ())), precision=_HI,
                               preferred_element_type=jnp.float32)


def _body(t_ref, sx_ref, g1_ref, bb1_ref, W1_ref, b1_ref, g2_ref, bb2_ref,
          W2_ref, b2_ref, orr_ref, rk_ref, *, rows, reg):
    R = rows
    x = t_ref[0]
    m = jnp.mean(x, axis=-1, keepdims=True)
    xc = x - m
    v = jnp.mean(xc * xc, axis=-1, keepdims=True)
    xn = xc / jnp.sqrt(v + 1e-5)
    xn = xn * g1_ref[...] + bb1_ref[...]
    h = jax.lax.dot_general(xn, W1_ref[...], (((1,), (1,)), ((), ())),
                            preferred_element_type=jnp.float32)
    h = jnp.maximum(h + b1_ref[...], 0.0)
    m2 = jnp.mean(h, axis=-1, keepdims=True)
    hc = h - m2
    v2 = jnp.mean(hc * hc, axis=-1, keepdims=True)
    h2 = hc / jnp.sqrt(v2 + 1e-5)
    h2 = h2 * g2_ref[...] + bb2_ref[...]
    sc_col = jnp.sum(h2 * W2_ref[...], axis=-1, keepdims=True) + b2_ref[...]

    row = jax.lax.broadcasted_iota(jnp.int32, (R, R), 0)
    col = jax.lax.broadcasted_iota(jnp.int32, (R, R), 1)
    eye = (row == col).astype(jnp.float32)

    sc_row = _dot(sc_col, eye, ((0,), (0,)))

    cmp = (sc_col < sc_row) | ((sc_col == sc_row) & (row < col))
    rank_row = jnp.sum(cmp.astype(jnp.int32), axis=0, keepdims=True)

    sx_row = sx_ref[0]
    sx_col = _dot(eye, sx_row, ((1,), (1,)))
    cmpx = (sx_col < sx_row) | ((sx_col == sx_row) & (row < col))
    rk_ref[0] = jnp.sum(cmpx.astype(jnp.int32), axis=0, keepdims=True)

    mn = jnp.min(sc_col)
    mx = jnp.max(sc_col)
    theta = (sc_row - mn) * ((100.0 / reg) / (mx - mn))

    P = (row == rank_row).astype(jnp.float32)
    a_row = _dot(theta, P, ((1,), (1,)))
    a_col = _dot(eye, a_row, ((1,), (1,)))

    colf = jax.lax.broadcasted_iota(jnp.int32, (1, R), 1).astype(jnp.float32)
    y_row = a_row - (colf + 1.0)
    c = y_row
    d = 1
    while d < R:
        c = c + jnp.concatenate(
            [jnp.zeros((1, d), jnp.float32), c[:, : R - d]], axis=1)
        d *= 2
    c_col = _dot(eye, c, ((1,), (1,)))
    y_col = a_col - (
        jax.lax.broadcasted_iota(jnp.int32, (R, 1), 0).astype(jnp.float32)
        + 1.0)

    seg_len = (col - row + 1).astype(jnp.float32)
    valid = col >= row
    avg = jnp.where(valid, (c - c_col + y_col) / seg_len, _BIG)
    d = 1
    while d < R:
        avg = jnp.minimum(avg, jnp.concatenate(
            [avg[:, d:], jnp.full((R, d), _BIG, jnp.float32)], axis=1))
        d *= 2
    m1 = jnp.where(row <= col, avg, -_BIG)
    d = 1
    while d < R:
        m1 = jnp.maximum(m1, jnp.concatenate(
            [jnp.full((d, R), -_BIG, jnp.float32), m1[: R - d, :]], axis=0))
        d *= 2
    sol = jnp.max(jnp.where(row == col, m1, -_BIG), axis=0, keepdims=True)
    out_asc = a_row - sol

    orr_ref[0] = _dot(out_asc, P, ((1,), (0,)))


def _ln(x, g, b, eps=1e-5):
    m = jnp.mean(x, axis=-1, keepdims=True)
    v = jnp.mean((x - m) ** 2, axis=-1, keepdims=True)
    return (x - m) / jnp.sqrt(v + eps) * g + b


def kernel(table, ln1_g, ln1_b, W1, b1, ln2_g, ln2_b, W2, b2):
    B, R = table.shape[0], table.shape[1]
    D = table.shape[2] * table.shape[3]
    H = W1.shape[0]
    t = table.reshape(B, R, D)
    cap = 64
    reg = 0.01

    hx = _ln(t, ln1_g, ln1_b)
    hx = jax.nn.relu(hx @ W1.T + b1)
    hx = _ln(hx, ln2_g, ln2_b)
    sx = (hx @ W2.T + b2).reshape(B, 1, R)

    body = functools.partial(_body, rows=R, reg=reg)
    orr, ranks = pl.pallas_call(
        body,
        grid=(B,),
        in_specs=[
            pl.BlockSpec((1, R, D), lambda b: (b, 0, 0)),
            pl.BlockSpec((1, 1, R), lambda b: (b, 0, 0)),
            pl.BlockSpec((1, D), lambda b: (0, 0)),
            pl.BlockSpec((1, D), lambda b: (0, 0)),
            pl.BlockSpec((H, D), lambda b: (0, 0)),
            pl.BlockSpec((1, H), lambda b: (0, 0)),
            pl.BlockSpec((1, H), lambda b: (0, 0)),
            pl.BlockSpec((1, H), lambda b: (0, 0)),
            pl.BlockSpec((1, H), lambda b: (0, 0)),
            pl.BlockSpec((1, 1), lambda b: (0, 0)),
        ],
        out_specs=[
            pl.BlockSpec((1, 1, R), lambda b: (b, 0, 0)),
            pl.BlockSpec((1, 1, R), lambda b: (b, 0, 0)),
        ],
        out_shape=[
            jax.ShapeDtypeStruct((B, 1, R), jnp.float32),
            jax.ShapeDtypeStruct((B, 1, R), jnp.int32),
        ],
        compiler_params=pltpu.CompilerParams(
            dimension_semantics=("arbitrary",)),
    )(t, sx, ln1_g.reshape(1, D), ln1_b.reshape(1, D), W1, b1.reshape(1, H),
      ln2_g.reshape(1, H), ln2_b.reshape(1, H), W2.reshape(1, H),
      b2.reshape(1, 1))

    original_ranks = orr.reshape(B, R, 1)
    rank_indices = jnp.broadcast_to(
        ranks[B - 1].reshape(1, R) // cap + 1, (B, R)).reshape(B, R, 1)
    return (original_ranks, rank_indices)
